# initial kernel scaffold (unmeasured)
import jax
import jax.numpy as jnp
from jax import lax
from jax.experimental import pallas as pl
from jax.experimental.pallas import tpu as pltpu

V_PER_SHARD = 4096


def kernel(ids, E):
    T = ids.shape[0]
    D = E.shape[1]

    my_x = lax.axis_index("x")
    local = ids - my_x * V_PER_SHARD
    owned = (local >= 0) & (local < V_PER_SHARD)
    safe = jnp.where(owned, local, 0)
    part = jnp.where(
        owned[:, None], jnp.take(E, safe, axis=0), 0.0
    ).astype(jnp.bfloat16)

    def body(part_ref, out_ref, comm_ref, send_sem, recv_sem):
        x = lax.axis_index("x")
        y = lax.axis_index("y")
        z = lax.axis_index("z")
        nbr = (1 - x, y, z)

        barrier_sem = pltpu.get_barrier_semaphore()
        pl.semaphore_signal(
            barrier_sem, inc=1, device_id=nbr,
            device_id_type=pl.DeviceIdType.MESH,
        )
        pl.semaphore_wait(barrier_sem, 1)

        rdma = pltpu.make_async_remote_copy(
            src_ref=part_ref,
            dst_ref=comm_ref,
            send_sem=send_sem,
            recv_sem=recv_sem,
            device_id=nbr,
            device_id_type=pl.DeviceIdType.MESH,
        )
        rdma.start()
        rdma.wait()

        out_ref[:, :] = (
            part_ref[:, :].astype(jnp.float32)
            + comm_ref[:, :].astype(jnp.float32)
        )

    return pl.pallas_call(
        body,
        out_shape=jax.ShapeDtypeStruct((T, D), jnp.float32),
        in_specs=[pl.BlockSpec(memory_space=pltpu.VMEM)],
        out_specs=pl.BlockSpec(memory_space=pltpu.VMEM),
        scratch_shapes=[
            pltpu.VMEM((T, D), jnp.bfloat16),
            pltpu.SemaphoreType.DMA,
            pltpu.SemaphoreType.DMA,
        ],
        compiler_params=pltpu.CompilerParams(collective_id=0),
    )(part)


# baseline (device time: 14269 ns/iter reference)
import functools

import jax
import jax.numpy as jnp
from jax import lax
from jax.experimental import pallas as pl
from jax.experimental.pallas import tpu as pltpu

V_PER_SHARD = 4096


def kernel(ids, E):
    T = ids.shape[0]
    D = E.shape[1]

    my_x = lax.axis_index("x")
    local = ids - my_x * V_PER_SHARD
    owned = (local >= 0) & (local < V_PER_SHARD)
    safe = jnp.where(owned, local, 0)
    part = jnp.where(
        owned[:, None], jnp.take(E, safe, axis=0), 0.0
    ).astype(jnp.bfloat16)

    def body(part_ref, out_ref, comm_ref, send_sem, recv_sem):
        x = lax.axis_index("x")
        y = lax.axis_index("y")
        z = lax.axis_index("z")
        nbr = (1 - x, y, z)

        comm_ref[0, :, :] = part_ref[:, :]

        barrier_sem = pltpu.get_barrier_semaphore()
        pl.semaphore_signal(
            barrier_sem, inc=1, device_id=nbr,
            device_id_type=pl.DeviceIdType.MESH,
        )
        pl.semaphore_wait(barrier_sem, 1)

        rdma = pltpu.make_async_remote_copy(
            src_ref=comm_ref.at[0],
            dst_ref=comm_ref.at[1],
            send_sem=send_sem,
            recv_sem=recv_sem,
            device_id=nbr,
            device_id_type=pl.DeviceIdType.MESH,
        )
        rdma.start()
        rdma.wait()

        out_ref[:, :] = (
            comm_ref[0, :, :].astype(jnp.float32)
            + comm_ref[1, :, :].astype(jnp.float32)
        )

        @functools.partial(
            pl.run_scoped, second_barrier=pltpu.SemaphoreType.REGULAR
        )
        def _(second_barrier):
            pl.semaphore_signal(
                second_barrier, inc=1, device_id=nbr,
                device_id_type=pl.DeviceIdType.MESH,
            )
            pl.semaphore_wait(second_barrier, 1)

    return pl.pallas_call(
        body,
        out_shape=jax.ShapeDtypeStruct((T, D), jnp.float32),
        in_specs=[pl.BlockSpec(memory_space=pltpu.VMEM)],
        out_specs=pl.BlockSpec(memory_space=pltpu.VMEM),
        scratch_shapes=[
            pltpu.VMEM((2, T, D), jnp.bfloat16),
            pltpu.SemaphoreType.DMA,
            pltpu.SemaphoreType.DMA,
        ],
        compiler_params=pltpu.CompilerParams(collective_id=0),
    )(part)


# device time: 13598 ns/iter; 1.0493x vs baseline; 1.0493x over previous
import functools

import jax
import jax.numpy as jnp
from jax import lax
from jax.experimental import pallas as pl
from jax.experimental.pallas import tpu as pltpu

V_PER_SHARD = 4096
NCHUNK = 32


def kernel(ids, E):
    T = ids.shape[0]
    D = E.shape[1]

    my_x = lax.axis_index("x")
    local = ids - my_x * V_PER_SHARD
    owned = (local >= 0) & (local < V_PER_SHARD)
    safe = jnp.where(owned, local, 0)
    part = jnp.take(E, safe, axis=0)
    mask = owned.astype(jnp.int32)[:, None]

    R = T // NCHUNK

    def body(mask_ref, part_ref, out_ref, sbuf_ref, rbuf_ref, send_sems, recv_sems):
        x = lax.axis_index("x")
        y = lax.axis_index("y")
        z = lax.axis_index("z")
        nbr = (1 - x, y, z)

        sbuf_ref[:, :] = part_ref[:, :].astype(jnp.bfloat16)

        barrier_sem = pltpu.get_barrier_semaphore()
        pl.semaphore_signal(
            barrier_sem, inc=1, device_id=nbr,
            device_id_type=pl.DeviceIdType.MESH,
        )
        pl.semaphore_wait(barrier_sem, 1)

        rdmas = [
            pltpu.make_async_remote_copy(
                src_ref=sbuf_ref.at[pl.ds(c * R, R)],
                dst_ref=rbuf_ref.at[pl.ds(c * R, R)],
                send_sem=send_sems.at[c],
                recv_sem=recv_sems.at[c],
                device_id=nbr,
                device_id_type=pl.DeviceIdType.MESH,
            )
            for c in range(NCHUNK)
        ]
        for r in rdmas:
            r.start()
        for r in rdmas:
            r.wait()

        m = mask_ref[:, :] > 0
        out_ref[:, :] = jnp.where(
            m, part_ref[:, :], rbuf_ref[:, :].astype(jnp.float32)
        )

        @functools.partial(
            pl.run_scoped, second_barrier=pltpu.SemaphoreType.REGULAR
        )
        def _(second_barrier):
            pl.semaphore_signal(
                second_barrier, inc=1, device_id=nbr,
                device_id_type=pl.DeviceIdType.MESH,
            )
            pl.semaphore_wait(second_barrier, 1)

    return pl.pallas_call(
        body,
        out_shape=jax.ShapeDtypeStruct((T, D), jnp.float32),
        in_specs=[
            pl.BlockSpec(memory_space=pltpu.VMEM),
            pl.BlockSpec(memory_space=pltpu.VMEM),
        ],
        out_specs=pl.BlockSpec(memory_space=pltpu.VMEM),
        scratch_shapes=[
            pltpu.VMEM((T, D), jnp.bfloat16),
            pltpu.VMEM((T, D), jnp.bfloat16),
            pltpu.SemaphoreType.DMA((NCHUNK,)),
            pltpu.SemaphoreType.DMA((NCHUNK,)),
        ],
        compiler_params=pltpu.CompilerParams(collective_id=0),
    )(mask, part)
